# Initial kernel scaffold; baseline (speedup 1.0000x reference)
import jax
import jax.numpy as jnp
from jax.experimental import pallas as pl


def kernel(h, edge_index, e, W_enc, b_enc, W0, b0, W1, b1, W2, b2, W3, b3, W_ro, b_ro):
    raise NotImplementedError("write your pallas kernel here")


# speedup vs baseline: 1.0310x; 1.0310x over previous
import functools

import jax
import jax.numpy as jnp
from jax.experimental import pallas as pl

N = 10000
D = 128
AVG_D_LOG = 3.5

_ROW_BLK = 1000


def _mm_kernel(x_ref, w_ref, b_ref, o_ref, *, relu):
    acc = jnp.dot(x_ref[...], w_ref[...], preferred_element_type=jnp.float32)
    acc = acc + b_ref[...][None, :]
    if relu:
        acc = jnp.maximum(acc, 0.0)
    o_ref[...] = acc


def _matmul_bias(x, w, b, relu=False):
    n, k = x.shape
    m = w.shape[1]
    grid = (n // _ROW_BLK,)
    return pl.pallas_call(
        functools.partial(_mm_kernel, relu=relu),
        grid=grid,
        in_specs=[
            pl.BlockSpec((_ROW_BLK, k), lambda i: (i, 0)),
            pl.BlockSpec((k, m), lambda i: (0, 0)),
            pl.BlockSpec((m,), lambda i: (0,)),
        ],
        out_specs=pl.BlockSpec((_ROW_BLK, m), lambda i: (i, 0)),
        out_shape=jax.ShapeDtypeStruct((n, m), jnp.float32),
    )(x, w, b)


def _combine_kernel(h_ref, s_ref, mx_ref, mn_ref, sq_ref, deg_ref, w_ref,
                    b_ref, o_ref):
    deg = deg_ref[...]
    degc = jnp.maximum(deg, 1.0)
    invd = 1.0 / degc
    has = deg > 0.0
    mean = s_ref[...] * invd
    msq = sq_ref[...] * invd
    std = jnp.sqrt(jnp.maximum(msq - mean * mean, 0.0) + 1e-5)
    mx = jnp.where(has, mx_ref[...], 0.0)
    mn = jnp.where(has, mn_ref[...], 0.0)
    agg = jnp.concatenate([mean, mx, mn, std], axis=1)
    ld = jnp.log(deg + 1.0)
    amp = ld / AVG_D_LOG
    att = AVG_D_LOG / jnp.maximum(ld, 1e-5)
    w = w_ref[...]
    acc = jnp.dot(h_ref[...], w[0:D], preferred_element_type=jnp.float32)
    acc += jnp.dot(agg, w[D:D + 512], preferred_element_type=jnp.float32)
    acc += jnp.dot(agg * amp, w[D + 512:D + 1024],
                   preferred_element_type=jnp.float32)
    acc += jnp.dot(agg * att, w[D + 1024:D + 1536],
                   preferred_element_type=jnp.float32)
    acc += b_ref[...][None, :]
    o_ref[...] = jnp.maximum(acc, 0.0)


def _layer_combine(h, s, mx, mn, sq, deg, w, b):
    grid = (N // _ROW_BLK,)
    blk = lambda i: (i, 0)
    return pl.pallas_call(
        _combine_kernel,
        grid=grid,
        in_specs=[
            pl.BlockSpec((_ROW_BLK, D), blk),
            pl.BlockSpec((_ROW_BLK, D), blk),
            pl.BlockSpec((_ROW_BLK, D), blk),
            pl.BlockSpec((_ROW_BLK, D), blk),
            pl.BlockSpec((_ROW_BLK, D), blk),
            pl.BlockSpec((_ROW_BLK, 1), blk),
            pl.BlockSpec((13 * D, D), lambda i: (0, 0)),
            pl.BlockSpec((D,), lambda i: (0,)),
        ],
        out_specs=pl.BlockSpec((_ROW_BLK, D), blk),
        out_shape=jax.ShapeDtypeStruct((N, D), jnp.float32),
    )(h, s, mx, mn, sq, deg, w, b)


def _aggregate(h, src, dst):
    m = h[src]
    s = jax.ops.segment_sum(m, dst, num_segments=N)
    sq = jax.ops.segment_sum(m * m, dst, num_segments=N)
    mx = jax.ops.segment_max(m, dst, num_segments=N)
    mn = -jax.ops.segment_max(-m, dst, num_segments=N)
    return s, mx, mn, sq


def kernel(h, edge_index, e, W_enc, b_enc, W0, b0, W1, b1, W2, b2, W3, b3,
           W_ro, b_ro):
    src = edge_index[0]
    dst = edge_index[1]
    deg = jax.ops.segment_sum(jnp.ones((src.shape[0],), jnp.float32), dst,
                              num_segments=N)
    deg2 = deg[:, None]
    h = _matmul_bias(h, W_enc, b_enc)
    for W, b in ((W0, b0), (W1, b1), (W2, b2), (W3, b3)):
        s, mx, mn, sq = _aggregate(h, src, dst)
        mx = jnp.where(deg2 > 0, mx, 0.0)
        mn = jnp.where(deg2 > 0, mn, 0.0)
        h = _layer_combine(h, s, mx, mn, sq, deg2, W, b)
    return _matmul_bias(h, W_ro, b_ro)
